# baseline (speedup 1.0000x reference)
import functools
import jax
import jax.numpy as jnp
from jax import lax
from jax.experimental import pallas as pl
from jax.experimental.pallas import tpu as pltpu
from jax.experimental.pallas import tpu_sc as plsc

EMB_DIM = 64
N_NEG = 20
LANES = 16
DCH = EMB_DIM // LANES


def _make_sc_partials(B):
    info = plsc.get_sparse_core_info()
    NC, NS = info.num_cores, info.num_subcores
    NW = NC * NS
    per_w = B // NW
    C = 32
    n_chunks = per_w // C
    NEG_I = C * N_NEG
    NSUB = NEG_I // 128

    mesh = plsc.VectorSubcoreMesh(core_axis_name="c", subcore_axis_name="s")

    @functools.partial(
        pl.kernel,
        mesh=mesh,
        compiler_params=pltpu.CompilerParams(use_tc_tiling_on_sc=True),
        out_type=[
            jax.ShapeDtypeStruct((B, LANES), jnp.float32),
            jax.ShapeDtypeStruct((B, LANES), jnp.float32),
        ],
        scratch_types=[
            pltpu.VMEM((C + LANES,), jnp.int32),
            pltpu.VMEM((C + LANES,), jnp.int32),
            pltpu.VMEM((NEG_I + LANES,), jnp.int32),
            pltpu.VMEM((C,), jnp.int32),
            pltpu.VMEM((C,), jnp.int32),
            pltpu.VMEM((NEG_I,), jnp.int32),
            pltpu.VMEM((C, 2 * EMB_DIM), jnp.float32),
            pltpu.VMEM((C, 2 * EMB_DIM), jnp.float32),
            pltpu.VMEM((NEG_I, 2 * EMB_DIM), jnp.float32),
            pltpu.VMEM((C, LANES), jnp.float32),
            pltpu.VMEM((C, LANES), jnp.float32),
            pltpu.SemaphoreType.DMA,
        ],
    )
    def sc_kernel(tgt_hbm, ctx_hbm, negidx_hbm, u_hbm, v_hbm,
                  pos_hbm, negp_hbm,
                  tgt_v, ctx_v, neg_iv, tgt_p, ctx_p, neg_p,
                  u_v, v_v, neg_v, posbuf, negbuf, sem):
        wid = lax.axis_index("s") * NC + lax.axis_index("c")
        base_w = wid * per_w

        def chunk_body(ci, carry):
            base = base_w + ci * C
            pltpu.sync_copy(tgt_hbm.at[pl.ds(base, C)],
                            tgt_v.at[pl.ds(0, C)])
            pltpu.sync_copy(ctx_hbm.at[pl.ds(base, C)],
                            ctx_v.at[pl.ds(0, C)])
            pltpu.sync_copy(negidx_hbm.at[pl.ds(base * N_NEG, NEG_I)],
                            neg_iv.at[pl.ds(0, NEG_I)])
            for j in range(C // LANES):
                sl = pl.ds(j * LANES, LANES)
                tgt_p[sl] = lax.shift_right_logical(tgt_v[sl], 1)
                ctx_p[sl] = lax.shift_right_logical(ctx_v[sl], 1)
            for j in range(NEG_I // LANES):
                sl = pl.ds(j * LANES, LANES)
                neg_p[sl] = lax.shift_right_logical(neg_iv[sl], 1)

            copies = [
                pltpu.async_copy(u_hbm.at[tgt_p], u_v, sem),
                pltpu.async_copy(v_hbm.at[ctx_p], v_v, sem),
            ]
            for j in range(NSUB):
                copies.append(pltpu.async_copy(
                    v_hbm.at[neg_p.at[pl.ds(j * 128, 128)]],
                    neg_v.at[pl.ds(j * 128, 128)], sem))
            for cp in copies:
                cp.wait()

            def elem_body(i, carry2):
                ot = (tgt_v[pl.ds(i, LANES)][0] & 1) * EMB_DIM
                oc = (ctx_v[pl.ds(i, LANES)][0] & 1) * EMB_DIM
                pos = None
                negp = None
                uks = []
                for kk in range(DCH):
                    uk = u_v[i, pl.ds(ot + kk * LANES, LANES)]
                    vk = v_v[i, pl.ds(oc + kk * LANES, LANES)]
                    uks.append(uk)
                    pk = uk * vk
                    pos = pk if pos is None else pos + pk
                accs = [None] * DCH
                for n in range(N_NEG):
                    on = (neg_iv[pl.ds(i * N_NEG + n, LANES)][0] & 1) * EMB_DIM
                    for kk in range(DCH):
                        r = neg_v[i * N_NEG + n, pl.ds(on + kk * LANES, LANES)]
                        accs[kk] = r if accs[kk] is None else accs[kk] + r
                for kk in range(DCH):
                    nk = uks[kk] * accs[kk]
                    negp = nk if negp is None else negp + nk
                posbuf[i, :] = pos
                negbuf[i, :] = negp
                return carry2

            lax.fori_loop(0, C, elem_body, 0)
            pltpu.sync_copy(posbuf, pos_hbm.at[pl.ds(base, C)])
            pltpu.sync_copy(negbuf, negp_hbm.at[pl.ds(base, C)])
            return carry

        lax.fori_loop(0, n_chunks, chunk_body, 0)

    return sc_kernel


def _tc_finish(pos_part, neg_part):
    def body(p_ref, n_ref, o_ref):
        p = jnp.sum(p_ref[...], axis=1)
        q = jnp.sum(n_ref[...], axis=1)

        def logsig(x):
            return jnp.minimum(x, 0.0) - jnp.log1p(jnp.exp(-jnp.abs(x)))

        loss = logsig(p) + logsig(-q)
        o_ref[...] = jnp.broadcast_to(-jnp.mean(loss), (1, 1))

    out = pl.pallas_call(
        body,
        out_shape=jax.ShapeDtypeStruct((1, 1), jnp.float32),
    )(pos_part, neg_part)
    return out[0, 0]


def kernel(target_word, context_word, neg_word, u_weight, v_weight):
    B = target_word.shape[0]
    neg_flat = neg_word.reshape(B * N_NEG)
    u2 = u_weight.reshape(-1, 2 * EMB_DIM)
    v2 = v_weight.reshape(-1, 2 * EMB_DIM)
    sc = _make_sc_partials(B)
    pos_part, neg_part = sc(target_word, context_word, neg_flat, u2, v2)
    return _tc_finish(pos_part, neg_part)


# speedup vs baseline: 1.1917x; 1.1917x over previous
import functools
import jax
import jax.numpy as jnp
from jax import lax
from jax.experimental import pallas as pl
from jax.experimental.pallas import tpu as pltpu
from jax.experimental.pallas import tpu_sc as plsc

EMB_DIM = 64
N_NEG = 20
LANES = 16
DCH = EMB_DIM // LANES
GRP = 8


def _make_sc_partials(B):
    info = plsc.get_sparse_core_info()
    NC, NS = info.num_cores, info.num_subcores
    NW = NC * NS
    per_w = B // NW
    C = 32
    n_chunks = per_w // C
    NEG_C = C * N_NEG
    NSUB = NEG_C // 128

    mesh = plsc.VectorSubcoreMesh(core_axis_name="c", subcore_axis_name="s")

    @functools.partial(
        pl.kernel,
        mesh=mesh,
        compiler_params=pltpu.CompilerParams(use_tc_tiling_on_sc=True),
        out_type=[
            jax.ShapeDtypeStruct((B, LANES), jnp.float32),
            jax.ShapeDtypeStruct((B, LANES), jnp.float32),
        ],
        scratch_types=[
            pltpu.VMEM((C + LANES,), jnp.int32),
            pltpu.VMEM((C + LANES,), jnp.int32),
            pltpu.VMEM((NEG_C + LANES,), jnp.int32),
            pltpu.VMEM((C,), jnp.int32),
            pltpu.VMEM((NEG_C,), jnp.int32),
            pltpu.VMEM((C * GRP, EMB_DIM), jnp.float32),
            pltpu.VMEM((C, 2 * EMB_DIM), jnp.float32),
            pltpu.VMEM((NEG_C, 2 * EMB_DIM), jnp.float32),
            pltpu.VMEM((C, LANES), jnp.float32),
            pltpu.VMEM((C, LANES), jnp.float32),
            pltpu.SemaphoreType.DMA,
        ],
    )
    def sc_kernel(tgt_hbm, ctx_hbm, negidx_hbm, u_hbm, v_hbm,
                  pos_hbm, negp_hbm,
                  tgt_v, ctx_v, neg_iv, ctx_p, neg_p,
                  u_b, v_b, neg_b, posbuf, negbuf, sem):
        wid = lax.axis_index("s") * NC + lax.axis_index("c")
        base_w = wid * per_w

        def chunk_body(ci, carry):
            base = base_w + ci * C
            pltpu.sync_copy(tgt_hbm.at[pl.ds(base, C)],
                            tgt_v.at[pl.ds(0, C)])
            pltpu.sync_copy(ctx_hbm.at[pl.ds(base, C)],
                            ctx_v.at[pl.ds(0, C)])
            pltpu.sync_copy(negidx_hbm.at[pl.ds(base * N_NEG, NEG_C)],
                            neg_iv.at[pl.ds(0, NEG_C)])
            for j in range(C // LANES):
                sl = pl.ds(j * LANES, LANES)
                ctx_p[sl] = lax.shift_right_logical(ctx_v[sl], 1)
            for j in range(NEG_C // LANES):
                sl = pl.ds(j * LANES, LANES)
                neg_p[sl] = lax.shift_right_logical(neg_iv[sl], 1)

            copies = [pltpu.async_copy(v_hbm.at[ctx_p], v_b, sem)]
            for j in range(NSUB):
                copies.append(pltpu.async_copy(
                    v_hbm.at[neg_p.at[pl.ds(j * 128, 128)]],
                    neg_b.at[pl.ds(j * 128, 128)], sem))
            for i in range(C):
                t = tgt_v[pl.ds(i, LANES)][0]
                gstart = pl.multiple_of(
                    lax.shift_left(lax.shift_right_logical(t, 3), 3), GRP)
                copies.append(pltpu.async_copy(
                    u_hbm.at[pl.ds(gstart, GRP)],
                    u_b.at[pl.ds(i * GRP, GRP)], sem))
            for cp in copies:
                cp.wait()

            def elem_body(i, carry2):
                rt = tgt_v[pl.ds(i, LANES)][0] & (GRP - 1)
                oc = (ctx_v[pl.ds(i, LANES)][0] & 1) * EMB_DIM
                pos = None
                negp = None
                uks = []
                for kk in range(DCH):
                    uk = u_b[i * GRP + rt, pl.ds(kk * LANES, LANES)]
                    vk = v_b[i, pl.ds(oc + kk * LANES, LANES)]
                    uks.append(uk)
                    pk = uk * vk
                    pos = pk if pos is None else pos + pk
                accs = [None] * DCH
                for n in range(N_NEG):
                    on = (neg_iv[pl.ds(i * N_NEG + n, LANES)][0] & 1) * EMB_DIM
                    for kk in range(DCH):
                        r = neg_b[i * N_NEG + n, pl.ds(on + kk * LANES, LANES)]
                        accs[kk] = r if accs[kk] is None else accs[kk] + r
                for kk in range(DCH):
                    nk = uks[kk] * accs[kk]
                    negp = nk if negp is None else negp + nk
                posbuf[i, :] = pos
                negbuf[i, :] = negp
                return carry2

            lax.fori_loop(0, C, elem_body, 0)
            pltpu.sync_copy(posbuf, pos_hbm.at[pl.ds(base, C)])
            pltpu.sync_copy(negbuf, negp_hbm.at[pl.ds(base, C)])
            return carry

        lax.fori_loop(0, n_chunks, chunk_body, 0)

    return sc_kernel


def _tc_finish(pos_part, neg_part):
    def body(p_ref, n_ref, o_ref):
        p = jnp.sum(p_ref[...], axis=1)
        q = jnp.sum(n_ref[...], axis=1)

        def logsig(x):
            return jnp.minimum(x, 0.0) - jnp.log1p(jnp.exp(-jnp.abs(x)))

        loss = logsig(p) + logsig(-q)
        o_ref[...] = jnp.broadcast_to(-jnp.mean(loss), (1, 1))

    out = pl.pallas_call(
        body,
        out_shape=jax.ShapeDtypeStruct((1, 1), jnp.float32),
    )(pos_part, neg_part)
    return out[0, 0]


def kernel(target_word, context_word, neg_word, u_weight, v_weight):
    B = target_word.shape[0]
    neg_flat = neg_word.reshape(B * N_NEG)
    v2 = v_weight.reshape(-1, 2 * EMB_DIM)
    sc = _make_sc_partials(B)
    pos_part, neg_part = sc(target_word, context_word, neg_flat,
                            u_weight, v2)
    return _tc_finish(pos_part, neg_part)
